# Initial kernel scaffold; baseline (speedup 1.0000x reference)
import jax
import jax.numpy as jnp
from jax.experimental import pallas as pl


def kernel(x, camera_matrix):
    raise NotImplementedError("write your pallas kernel here")


# speedup vs baseline: 2.9175x; 2.9175x over previous
import functools

import jax
import jax.numpy as jnp
from jax import lax
from jax.experimental import pallas as pl
from jax.experimental.pallas import tpu as pltpu
from jax.experimental.pallas import tpu_sc as plsc

OFF_THRESH = 0.5
INF = 1e10

H = 384
W = 384
B = 2
HP = H + 2
WBUF = 416
ROWS_PER_WORKER = 24
S1_ROWS = ROWS_PER_WORKER + 2
Z_ROWS = S1_ROWS + 2
NC = 2
NS = 16


def _sc_body(xp_hbm, c_hbm, out_hbm,
             zbuf, xbuf, ybuf, mbuf, ixz, irm, ox, oy, oz, om, cbuf):
    wid = lax.axis_index("s") * NC + lax.axis_index("c")
    b = wid // NS
    wi = wid % NS
    r0 = ROWS_PER_WORKER * wi
    ch = b * 4

    pltpu.sync_copy(c_hbm, cbuf)
    pltpu.sync_copy(xp_hbm.at[pl.ds((ch + 0) * 388 + r0 + 1, S1_ROWS), :], xbuf)
    pltpu.sync_copy(xp_hbm.at[pl.ds((ch + 1) * 388 + r0 + 1, S1_ROWS), :], ybuf)
    pltpu.sync_copy(xp_hbm.at[pl.ds((ch + 2) * 388 + r0, Z_ROWS), :], zbuf)
    pltpu.sync_copy(xp_hbm.at[pl.ds((ch + 3) * 388 + r0 + 1, S1_ROWS), :], mbuf)

    a = [cbuf[i, :] for i in range(9)]
    iota = lax.iota(jnp.int32, 16).astype(jnp.float32)
    inf_v = jnp.full((16,), INF, jnp.float32)
    zero_v = jnp.zeros((16,), jnp.float32)

    def s1_row(rr, _):
        vf = (r0 + rr - 1).astype(jnp.float32)

        def s1_chunk(j, _):
            c0 = 16 * j
            bc = c0 + 1
            u = iota + (c0 - 1).astype(jnp.float32)
            xc = xbuf[rr, pl.ds(bc, 16)]
            yc = ybuf[rr, pl.ds(bc, 16)]
            zc = zbuf[rr + 1, pl.ds(bc, 16)]
            mc = mbuf[rr, pl.ds(bc, 16)]
            best_d = None
            best_k = None
            for k in range(9):
                dr, dc = k // 3 - 1, k % 3 - 1
                zn = zbuf[rr + 1 + dr, pl.ds(bc + dc, 16)]
                un = u + float(dc)
                vn = jnp.broadcast_to(vf + float(dr), (16,))
                rx = a[0] * un + a[1] * vn + a[2]
                ry = a[3] * un + a[4] * vn + a[5]
                rz = a[6] * un + a[7] * vn + a[8]
                dx = rx * zn - xc
                dy = ry * zn - yc
                dz = rz * zn - zc
                d = dx * dx + dy * dy + dz * dz
                if best_d is None:
                    best_d = d
                    best_k = jnp.zeros((16,), jnp.int32)
                else:
                    m = d < best_d
                    best_d = jnp.where(m, d, best_d)
                    best_k = jnp.where(m, k, best_k)
            rmask = mc > OFF_THRESH
            zmask = zc > 0.0
            ixz[rr, pl.ds(bc, 16)] = jnp.where(
                rmask, jnp.where(zmask, best_k, 4), 9)
            irm[rr, pl.ds(bc, 16)] = jnp.where(rmask & zmask, best_k, 4)
            return 0

        lax.fori_loop(0, 25, s1_chunk, 0)
        return 0

    lax.fori_loop(0, S1_ROWS, s1_row, 0)

    def s2_row(rr2, _):
        xr = rr2 + 1

        def s2_chunk(j2, _):
            b0 = 16 * j2 + 2
            ixz_c = ixz[xr, pl.ds(b0, 16)]
            irm_c = irm[xr, pl.ds(b0, 16)]
            xc = xbuf[xr, pl.ds(b0, 16)]
            yc = ybuf[xr, pl.ds(b0, 16)]
            zc = zbuf[xr + 1, pl.ds(b0, 16)]
            mc = mbuf[xr, pl.ds(b0, 16)]
            c4 = ixz_c == 4
            bx = jnp.where(c4, xc, zero_v)
            by = jnp.where(c4, yc, zero_v)
            bz = jnp.where(c4, zc, zero_v)
            brm = jnp.where(irm_c == 4, mc, zero_v)
            best = inf_v
            for k in range(9):
                s, t = k // 3, k % 3
                ro = -(s - 1)
                co = -(t - 1)
                if ro == 0 and co == 0:
                    ixz_q, zq, xq, yq, mq = ixz_c, zc, xc, yc, mc
                else:
                    ixz_q = ixz[xr + ro, pl.ds(b0 + co, 16)]
                    zq = zbuf[xr + 1 + ro, pl.ds(b0 + co, 16)]
                    xq = xbuf[xr + ro, pl.ds(b0 + co, 16)]
                    yq = ybuf[xr + ro, pl.ds(b0 + co, 16)]
                    mq = mbuf[xr + ro, pl.ds(b0 + co, 16)]
                cand = jnp.where(ixz_q == k, zq, zero_v)
                cand = jnp.where(cand > 0.0, cand, inf_v)
                m = cand < best
                best = jnp.where(m, cand, best)
                bx = jnp.where(m, xq, bx)
                by = jnp.where(m, yq, by)
                bz = jnp.where(m, zq, bz)
                brm = jnp.where(m, mq, brm)
            ox[rr2, pl.ds(16 * j2, 16)] = bx
            oy[rr2, pl.ds(16 * j2, 16)] = by
            oz[rr2, pl.ds(16 * j2, 16)] = bz
            om[rr2, pl.ds(16 * j2, 16)] = brm
            return 0

        lax.fori_loop(0, W // 16, s2_chunk, 0)
        return 0

    lax.fori_loop(0, ROWS_PER_WORKER, s2_row, 0)

    base = b * 4 * H + r0
    pltpu.sync_copy(ox, out_hbm.at[pl.ds(base + 0 * H, ROWS_PER_WORKER), :])
    pltpu.sync_copy(oy, out_hbm.at[pl.ds(base + 1 * H, ROWS_PER_WORKER), :])
    pltpu.sync_copy(oz, out_hbm.at[pl.ds(base + 2 * H, ROWS_PER_WORKER), :])
    pltpu.sync_copy(om, out_hbm.at[pl.ds(base + 3 * H, ROWS_PER_WORKER), :])


_smap_sc = functools.partial(
    pl.kernel,
    out_type=jax.ShapeDtypeStruct((B * 4 * H, W), jnp.float32),
    mesh=plsc.VectorSubcoreMesh(core_axis_name="c", subcore_axis_name="s"),
    compiler_params=pltpu.CompilerParams(use_tc_tiling_on_sc=False),
    scratch_types=[
        pltpu.VMEM((Z_ROWS, WBUF), jnp.float32),
        pltpu.VMEM((S1_ROWS, WBUF), jnp.float32),
        pltpu.VMEM((S1_ROWS, WBUF), jnp.float32),
        pltpu.VMEM((S1_ROWS, WBUF), jnp.float32),
        pltpu.VMEM((S1_ROWS, WBUF), jnp.int32),
        pltpu.VMEM((S1_ROWS, WBUF), jnp.int32),
        pltpu.VMEM((ROWS_PER_WORKER, W), jnp.float32),
        pltpu.VMEM((ROWS_PER_WORKER, W), jnp.float32),
        pltpu.VMEM((ROWS_PER_WORKER, W), jnp.float32),
        pltpu.VMEM((ROWS_PER_WORKER, W), jnp.float32),
        pltpu.VMEM((9, 16), jnp.float32),
    ],
)(_sc_body)


def kernel(x, camera_matrix):
    k_inv = jnp.linalg.inv(camera_matrix)
    consts = jnp.repeat(k_inv.reshape(9, 1), 16, axis=1)
    xp = jnp.pad(x, ((0, 0), (0, 0), (2, 2), (2, WBUF - W - 2)))
    xp = xp.reshape(B * 4 * (HP + 2), WBUF)
    out = _smap_sc(xp, consts)
    return out.reshape(B, 4, H, W)
